# baseline (speedup 1.0000x reference)
import jax
import jax.numpy as jnp
from jax.experimental import pallas as pl
from jax.experimental.pallas import tpu as pltpu

DIM = 1024
N_WORK = 32
N_PERSIST = 64
N_LONG = 256
N_TOT = N_WORK + N_PERSIST + N_LONG
M_PAD = 384
TILE = 512


_N_SUB = 4


def _body(q_ref, wc_ref, bc_ref, cmt_ref, cm_ref, o_ref):
    col = jax.lax.broadcasted_iota(jnp.int32, (1, M_PAD), 1)
    m0 = col < N_WORK
    m1 = (col >= N_WORK) & (col < N_WORK + N_PERSIST)
    m2 = (col >= N_WORK + N_PERSIST) & (col < N_TOT)
    neg = jnp.float32(-jnp.inf)
    third = jnp.float32(1.0 / 3.0)
    wc = wc_ref[...]
    bc = bc_ref[...]
    cmt = cmt_ref[...]
    cm = cm_ref[...]
    sub = TILE // _N_SUB
    for k in range(_N_SUB):
        rows = pl.ds(k * sub, sub)
        q = q_ref[rows, :].astype(jnp.bfloat16)
        qp = jnp.dot(q, wc, preferred_element_type=jnp.float32) + bc
        s = jnp.dot(qp.astype(jnp.bfloat16), cmt,
                    preferred_element_type=jnp.float32)
        mx0 = jnp.max(jnp.where(m0, s, neg), axis=-1, keepdims=True)
        mx1 = jnp.max(jnp.where(m1, s, neg), axis=-1, keepdims=True)
        mx2 = jnp.max(jnp.where(m2, s, neg), axis=-1, keepdims=True)
        mx_sel = jnp.where(m0, mx0, jnp.where(m1, mx1, mx2))
        e = jnp.where(col < N_TOT, jnp.exp(s - mx_sel), 0.0)
        d0 = jnp.sum(jnp.where(m0, e, 0.0), axis=-1, keepdims=True)
        d1 = jnp.sum(jnp.where(m1, e, 0.0), axis=-1, keepdims=True)
        d2 = jnp.sum(jnp.where(m2, e, 0.0), axis=-1, keepdims=True)
        r = jnp.where(m0, third / d0, jnp.where(m1, third / d1, third / d2))
        probs = e * r
        o_ref[rows, :] = jnp.dot(probs.astype(jnp.bfloat16), cm,
                                 preferred_element_type=jnp.float32)


@jax.jit
def kernel(query_states, Wc, bc, working_memory, persistent_memory,
           long_term_buffer):
    B, S, D = query_states.shape
    q2 = query_states.reshape(B * S, D)
    cmem = jnp.concatenate(
        [working_memory[0], persistent_memory[0], long_term_buffer[0],
         jnp.zeros((M_PAD - N_TOT, D), dtype=query_states.dtype)], axis=0)
    scale = 1.0 / jnp.sqrt(jnp.float32(DIM))
    cmt = (cmem.T * scale).astype(jnp.bfloat16)
    cm16 = cmem.astype(jnp.bfloat16)
    wc16 = Wc.astype(jnp.bfloat16)
    bc2 = bc.reshape(1, D)

    grid = (B * S // TILE,)
    out = pl.pallas_call(
        _body,
        grid=grid,
        in_specs=[
            pl.BlockSpec((TILE, D), lambda i: (i, 0)),
            pl.BlockSpec((D, D), lambda i: (0, 0)),
            pl.BlockSpec((1, D), lambda i: (0, 0)),
            pl.BlockSpec((D, M_PAD), lambda i: (0, 0)),
            pl.BlockSpec((M_PAD, D), lambda i: (0, 0)),
        ],
        out_specs=pl.BlockSpec((TILE, D), lambda i: (i, 0)),
        out_shape=jax.ShapeDtypeStruct((B * S, D), jnp.float32),
        compiler_params=pltpu.CompilerParams(
            dimension_semantics=("parallel",)),
    )(q2, wc16, bc2, cmt, cm16)
    return out.reshape(B, S, D)


# speedup vs baseline: 1.7992x; 1.7992x over previous
import jax
import jax.numpy as jnp
from jax.experimental import pallas as pl
from jax.experimental.pallas import tpu as pltpu

DIM = 1024
N_WORK = 32
N_PERSIST = 64
N_LONG = 256
N_TOT = N_WORK + N_PERSIST + N_LONG
M_PAD = 384
TILE = 512


def _fold_body(wc_ref, bc_ref, cmt_ref, w2_ref, b2_ref):
    scale = 1.0 / jnp.sqrt(jnp.float32(DIM))
    cmt = cmt_ref[...]
    w2 = jnp.dot(wc_ref[...], cmt, preferred_element_type=jnp.float32)
    w2_ref[...] = (w2 * scale).astype(jnp.bfloat16)
    b2 = jnp.dot(bc_ref[...], cmt, preferred_element_type=jnp.float32)
    b2_ref[...] = b2 * scale


def _body(q_ref, w2_ref, b2_ref, cm_ref, o_ref):
    col = jax.lax.broadcasted_iota(jnp.int32, (1, M_PAD), 1)
    m0 = col < N_WORK
    m1 = (col >= N_WORK) & (col < N_WORK + N_PERSIST)
    m2 = (col >= N_WORK + N_PERSIST) & (col < N_TOT)
    neg = jnp.float32(-jnp.inf)
    third = jnp.float32(1.0 / 3.0)

    q = q_ref[...].astype(jnp.bfloat16)
    s = jnp.dot(q, w2_ref[...], preferred_element_type=jnp.float32)
    s = s + b2_ref[...]
    mx0 = jnp.max(jnp.where(m0, s, neg), axis=-1, keepdims=True)
    mx1 = jnp.max(jnp.where(m1, s, neg), axis=-1, keepdims=True)
    mx2 = jnp.max(jnp.where(m2, s, neg), axis=-1, keepdims=True)
    mx_sel = jnp.where(m0, mx0, jnp.where(m1, mx1, mx2))
    e = jnp.where(col < N_TOT, jnp.exp(s - mx_sel), 0.0)
    d0 = jnp.sum(jnp.where(m0, e, 0.0), axis=-1, keepdims=True)
    d1 = jnp.sum(jnp.where(m1, e, 0.0), axis=-1, keepdims=True)
    d2 = jnp.sum(jnp.where(m2, e, 0.0), axis=-1, keepdims=True)
    r = jnp.where(m0, third / d0, jnp.where(m1, third / d1, third / d2))
    probs = e * r
    o_ref[...] = jnp.dot(probs.astype(jnp.bfloat16), cm_ref[...],
                         preferred_element_type=jnp.float32)


@jax.jit
def kernel(query_states, Wc, bc, working_memory, persistent_memory,
           long_term_buffer):
    B, S, D = query_states.shape
    q2 = query_states.reshape(B * S, D)
    cmem = jnp.concatenate(
        [working_memory[0], persistent_memory[0], long_term_buffer[0],
         jnp.zeros((M_PAD - N_TOT, D), dtype=query_states.dtype)], axis=0)
    cmt = cmem.T
    cm16 = cmem.astype(jnp.bfloat16)
    bc2 = bc.reshape(1, D)

    w2, b2 = pl.pallas_call(
        _fold_body,
        out_shape=(jax.ShapeDtypeStruct((D, M_PAD), jnp.bfloat16),
                   jax.ShapeDtypeStruct((1, M_PAD), jnp.float32)),
    )(Wc, bc2, cmt)

    grid = (B * S // TILE,)
    out = pl.pallas_call(
        _body,
        grid=grid,
        in_specs=[
            pl.BlockSpec((TILE, D), lambda i: (i, 0)),
            pl.BlockSpec((D, M_PAD), lambda i: (0, 0)),
            pl.BlockSpec((1, M_PAD), lambda i: (0, 0)),
            pl.BlockSpec((M_PAD, D), lambda i: (0, 0)),
        ],
        out_specs=pl.BlockSpec((TILE, D), lambda i: (i, 0)),
        out_shape=jax.ShapeDtypeStruct((B * S, D), jnp.float32),
        compiler_params=pltpu.CompilerParams(
            dimension_semantics=("parallel",)),
    )(q2, w2, b2, cm16)
    return out.reshape(B, S, D)


# speedup vs baseline: 2.1669x; 1.2044x over previous
import jax
import jax.numpy as jnp
from jax.experimental import pallas as pl
from jax.experimental.pallas import tpu as pltpu

DIM = 1024
N_WORK = 32
N_PERSIST = 64
N_LONG = 256
N_TOT = N_WORK + N_PERSIST + N_LONG
M_PAD = 384
TILE = 1024


def _fold_body(wc_ref, bc_ref, cmt_ref, w2_ref, b2_ref):
    scale = 1.0 / jnp.sqrt(jnp.float32(DIM))
    cmt = cmt_ref[...]
    w2 = jnp.dot(wc_ref[...], cmt, preferred_element_type=jnp.float32)
    w2_ref[...] = (w2 * scale).astype(jnp.bfloat16)
    b2 = jnp.dot(bc_ref[...], cmt, preferred_element_type=jnp.float32)
    b2_ref[...] = b2 * scale


def _body(q_ref, w2_ref, b2_ref, cm_ref, o_ref):
    col = jax.lax.broadcasted_iota(jnp.int32, (1, M_PAD), 1)
    m0 = col < N_WORK
    m1 = (col >= N_WORK) & (col < N_WORK + N_PERSIST)
    m2 = (col >= N_WORK + N_PERSIST) & (col < N_TOT)
    neg = jnp.float32(-jnp.inf)
    third = jnp.float32(1.0 / 3.0)

    q = q_ref[...].astype(jnp.bfloat16)
    s = jnp.dot(q, w2_ref[...], preferred_element_type=jnp.float32)
    s = s + b2_ref[...]
    mx0 = jnp.max(jnp.where(m0, s, neg), axis=-1, keepdims=True)
    mx1 = jnp.max(jnp.where(m1, s, neg), axis=-1, keepdims=True)
    mx2 = jnp.max(jnp.where(m2, s, neg), axis=-1, keepdims=True)
    mx_sel = jnp.where(m0, mx0, jnp.where(m1, mx1, mx2))
    e = jnp.where(col < N_TOT, jnp.exp(s - mx_sel), 0.0)
    d0 = jnp.sum(jnp.where(m0, e, 0.0), axis=-1, keepdims=True)
    d1 = jnp.sum(jnp.where(m1, e, 0.0), axis=-1, keepdims=True)
    d2 = jnp.sum(jnp.where(m2, e, 0.0), axis=-1, keepdims=True)
    r = jnp.where(m0, third / d0, jnp.where(m1, third / d1, third / d2))
    probs = e * r
    o_ref[...] = jnp.dot(probs.astype(jnp.bfloat16), cm_ref[...],
                         preferred_element_type=jnp.float32)


@jax.jit
def kernel(query_states, Wc, bc, working_memory, persistent_memory,
           long_term_buffer):
    B, S, D = query_states.shape
    q2 = query_states.reshape(B * S, D)
    cmem = jnp.concatenate(
        [working_memory[0], persistent_memory[0], long_term_buffer[0],
         jnp.zeros((M_PAD - N_TOT, D), dtype=query_states.dtype)], axis=0)
    cmt = cmem.T
    cm16 = cmem.astype(jnp.bfloat16)
    bc2 = bc.reshape(1, D)

    w2, b2 = pl.pallas_call(
        _fold_body,
        out_shape=(jax.ShapeDtypeStruct((D, M_PAD), jnp.bfloat16),
                   jax.ShapeDtypeStruct((1, M_PAD), jnp.float32)),
    )(Wc, bc2, cmt)

    grid = (B * S // TILE,)
    out = pl.pallas_call(
        _body,
        grid=grid,
        in_specs=[
            pl.BlockSpec((TILE, D), lambda i: (i, 0)),
            pl.BlockSpec((D, M_PAD), lambda i: (0, 0)),
            pl.BlockSpec((1, M_PAD), lambda i: (0, 0)),
            pl.BlockSpec((M_PAD, D), lambda i: (0, 0)),
        ],
        out_specs=pl.BlockSpec((TILE, D), lambda i: (i, 0)),
        out_shape=jax.ShapeDtypeStruct((B * S, D), jnp.float32),
        compiler_params=pltpu.CompilerParams(
            dimension_semantics=("parallel",)),
    )(q2, w2, b2, cm16)
    return out.reshape(B, S, D)


# speedup vs baseline: 2.3639x; 1.0909x over previous
import jax
import jax.numpy as jnp
from jax.experimental import pallas as pl
from jax.experimental.pallas import tpu as pltpu

DIM = 1024
N_WORK = 32
N_PERSIST = 64
N_LONG = 256
N_TOT = N_WORK + N_PERSIST + N_LONG
M_PAD = 384
TILE = 2048


def _fold_body(wc_ref, bc_ref, cmt_ref, w2_ref, b2_ref):
    scale = 1.0 / jnp.sqrt(jnp.float32(DIM))
    cmt = cmt_ref[...]
    w2 = jnp.dot(wc_ref[...], cmt, preferred_element_type=jnp.float32)
    w2_ref[...] = (w2 * scale).astype(jnp.bfloat16)
    b2 = jnp.dot(bc_ref[...], cmt, preferred_element_type=jnp.float32)
    b2_ref[...] = b2 * scale


def _body(q_ref, w2_ref, b2_ref, cm_ref, o_ref):
    col = jax.lax.broadcasted_iota(jnp.int32, (1, M_PAD), 1)
    m0 = col < N_WORK
    m1 = (col >= N_WORK) & (col < N_WORK + N_PERSIST)
    m2 = (col >= N_WORK + N_PERSIST) & (col < N_TOT)
    neg = jnp.float32(-jnp.inf)
    third = jnp.float32(1.0 / 3.0)

    q = q_ref[...].astype(jnp.bfloat16)
    s = jnp.dot(q, w2_ref[...], preferred_element_type=jnp.float32)
    s = s + b2_ref[...]
    mx0 = jnp.max(jnp.where(m0, s, neg), axis=-1, keepdims=True)
    mx1 = jnp.max(jnp.where(m1, s, neg), axis=-1, keepdims=True)
    mx2 = jnp.max(jnp.where(m2, s, neg), axis=-1, keepdims=True)
    mx_sel = jnp.where(m0, mx0, jnp.where(m1, mx1, mx2))
    e = jnp.where(col < N_TOT, jnp.exp(s - mx_sel), 0.0)
    d0 = jnp.sum(jnp.where(m0, e, 0.0), axis=-1, keepdims=True)
    d1 = jnp.sum(jnp.where(m1, e, 0.0), axis=-1, keepdims=True)
    d2 = jnp.sum(jnp.where(m2, e, 0.0), axis=-1, keepdims=True)
    r = jnp.where(m0, third / d0, jnp.where(m1, third / d1, third / d2))
    probs = e * r
    o_ref[...] = jnp.dot(probs.astype(jnp.bfloat16), cm_ref[...],
                         preferred_element_type=jnp.float32)


@jax.jit
def kernel(query_states, Wc, bc, working_memory, persistent_memory,
           long_term_buffer):
    B, S, D = query_states.shape
    q2 = query_states.reshape(B * S, D)
    cmem = jnp.concatenate(
        [working_memory[0], persistent_memory[0], long_term_buffer[0],
         jnp.zeros((M_PAD - N_TOT, D), dtype=query_states.dtype)], axis=0)
    cmt = cmem.T
    cm16 = cmem.astype(jnp.bfloat16)
    bc2 = bc.reshape(1, D)

    w2, b2 = pl.pallas_call(
        _fold_body,
        out_shape=(jax.ShapeDtypeStruct((D, M_PAD), jnp.bfloat16),
                   jax.ShapeDtypeStruct((1, M_PAD), jnp.float32)),
    )(Wc, bc2, cmt)

    grid = (B * S // TILE,)
    out = pl.pallas_call(
        _body,
        grid=grid,
        in_specs=[
            pl.BlockSpec((TILE, D), lambda i: (i, 0)),
            pl.BlockSpec((D, M_PAD), lambda i: (0, 0)),
            pl.BlockSpec((1, M_PAD), lambda i: (0, 0)),
            pl.BlockSpec((M_PAD, D), lambda i: (0, 0)),
        ],
        out_specs=pl.BlockSpec((TILE, D), lambda i: (i, 0)),
        out_shape=jax.ShapeDtypeStruct((B * S, D), jnp.float32),
        compiler_params=pltpu.CompilerParams(
            dimension_semantics=("parallel",)),
    )(q2, w2, b2, cm16)
    return out.reshape(B, S, D)


# speedup vs baseline: 2.5267x; 1.0689x over previous
import jax
import jax.numpy as jnp
from jax.experimental import pallas as pl
from jax.experimental.pallas import tpu as pltpu

DIM = 1024
N_WORK = 32
N_PERSIST = 64
N_LONG = 256
N_TOT = N_WORK + N_PERSIST + N_LONG
M_PAD = 384
TILE = 2048


def _body(q_ref, wc_ref, bc_ref, wm_ref, pm_ref, lm_ref, o_ref,
          w2_ref, b2_ref, cm_ref):
    @pl.when(pl.program_id(0) == 0)
    def _fold():
        cmf = jnp.concatenate(
            [wm_ref[...], pm_ref[...], lm_ref[...],
             jnp.zeros((M_PAD - N_TOT, DIM), jnp.float32)], axis=0)
        cm_ref[...] = cmf.astype(jnp.bfloat16)
        scale = 1.0 / jnp.sqrt(jnp.float32(DIM))
        cmt = cmf.T
        w2 = jnp.dot(wc_ref[...], cmt, preferred_element_type=jnp.float32)
        w2_ref[...] = (w2 * scale).astype(jnp.bfloat16)
        b2 = jnp.dot(bc_ref[...], cmt, preferred_element_type=jnp.float32)
        b2_ref[...] = b2 * scale

    col = jax.lax.broadcasted_iota(jnp.int32, (1, M_PAD), 1)
    m0 = col < N_WORK
    m1 = (col >= N_WORK) & (col < N_WORK + N_PERSIST)
    m2 = (col >= N_WORK + N_PERSIST) & (col < N_TOT)
    neg = jnp.float32(-jnp.inf)
    third = jnp.float32(1.0 / 3.0)

    q = q_ref[...].astype(jnp.bfloat16)
    s = jnp.dot(q, w2_ref[...], preferred_element_type=jnp.float32)
    s = s + b2_ref[...]
    mx0 = jnp.max(jnp.where(m0, s, neg), axis=-1, keepdims=True)
    mx1 = jnp.max(jnp.where(m1, s, neg), axis=-1, keepdims=True)
    mx2 = jnp.max(jnp.where(m2, s, neg), axis=-1, keepdims=True)
    mx_sel = jnp.where(m0, mx0, jnp.where(m1, mx1, mx2))
    e = jnp.where(col < N_TOT, jnp.exp(s - mx_sel), 0.0)
    d0 = jnp.sum(jnp.where(m0, e, 0.0), axis=-1, keepdims=True)
    d1 = jnp.sum(jnp.where(m1, e, 0.0), axis=-1, keepdims=True)
    d2 = jnp.sum(jnp.where(m2, e, 0.0), axis=-1, keepdims=True)
    r = jnp.where(m0, third / d0, jnp.where(m1, third / d1, third / d2))
    probs = e * r
    o_ref[...] = jnp.dot(probs.astype(jnp.bfloat16), cm_ref[...],
                         preferred_element_type=jnp.float32)


@jax.jit
def kernel(query_states, Wc, bc, working_memory, persistent_memory,
           long_term_buffer):
    B, S, D = query_states.shape
    q2 = query_states.reshape(B * S, D)
    bc2 = bc.reshape(1, D)

    grid = (B * S // TILE,)
    const = lambda i: (0, 0)
    out = pl.pallas_call(
        _body,
        grid=grid,
        in_specs=[
            pl.BlockSpec((TILE, D), lambda i: (i, 0)),
            pl.BlockSpec((D, D), const),
            pl.BlockSpec((1, D), const),
            pl.BlockSpec((N_WORK, D), const),
            pl.BlockSpec((N_PERSIST, D), const),
            pl.BlockSpec((N_LONG, D), const),
        ],
        out_specs=pl.BlockSpec((TILE, D), lambda i: (i, 0)),
        out_shape=jax.ShapeDtypeStruct((B * S, D), jnp.float32),
        scratch_shapes=[
            pltpu.VMEM((D, M_PAD), jnp.bfloat16),
            pltpu.VMEM((1, M_PAD), jnp.float32),
            pltpu.VMEM((M_PAD, D), jnp.bfloat16),
        ],
        compiler_params=pltpu.CompilerParams(
            dimension_semantics=("arbitrary",)),
    )(q2, Wc, bc2, working_memory[0], persistent_memory[0],
      long_term_buffer[0])
    return out.reshape(B, S, D)
